# Initial kernel scaffold; baseline (speedup 1.0000x reference)
import jax
import jax.numpy as jnp
from jax.experimental import pallas as pl


def kernel(indices, table):
    raise NotImplementedError("write your pallas kernel here")


# speedup vs baseline: 4.1202x; 4.1202x over previous
import functools
import jax
import jax.numpy as jnp
from jax import lax
from jax.experimental import pallas as pl
from jax.experimental.pallas import tpu as pltpu
from jax.experimental.pallas import tpu_sc as plsc

_NUM_SETS = 26
_SET_LEN = 10
_EMBED_DIM = 64
_BATCH = 16384
_NUM_ROWS = _NUM_SETS * _SET_LEN

_L = 16
_NC, _NS = 2, 16
_NW = _NC * _NS
_E_TOTAL = _BATCH * _NUM_SETS
_E_PER_W = _E_TOTAL // _NW
_CHUNK_E = 416
_N_CHUNKS = _E_PER_W // _CHUNK_E
_GROUPS = _CHUNK_E // _L

_mesh = plsc.VectorSubcoreMesh(core_axis_name="c", subcore_axis_name="s")


@functools.partial(
    pl.kernel,
    out_type=jax.ShapeDtypeStruct((_E_TOTAL * _EMBED_DIM,), jnp.float32),
    mesh=_mesh,
    compiler_params=pltpu.CompilerParams(needs_layout_passes=False),
    scratch_types=[
        pltpu.VMEM((_NUM_ROWS * _EMBED_DIM,), jnp.float32),
        pltpu.VMEM((_CHUNK_E,), jnp.int32),
        pltpu.VMEM((_CHUNK_E * _EMBED_DIM,), jnp.float32),
    ],
)
def _sc_gather(idx_hbm, table_hbm, out_hbm, table_v, idx_v, rows_v):
    wid = lax.axis_index("s") * _NC + lax.axis_index("c")
    wbase = wid * _E_PER_W

    pltpu.sync_copy(table_hbm, table_v)

    lane = lax.iota(jnp.int32, _L)

    def do_chunk(chunk, _):
        cbase = wbase + chunk * _CHUNK_E
        idx_b = idx_v
        rows_b = rows_v
        pltpu.sync_copy(idx_hbm.at[pl.ds(cbase, _CHUNK_E)], idx_b)

        def do_group(g, _):
            pos = g * _L + lane
            offs = lax.rem(pos, _NUM_SETS) * _SET_LEN
            rows = idx_b[pl.ds(g * _L, _L)] + offs
            src_base = rows * _EMBED_DIM
            dst_base = pos * _EMBED_DIM
            for c in range(_EMBED_DIM):
                v = plsc.load_gather(table_v, [src_base + c])
                plsc.store_scatter(rows_b, [dst_base + c], v)
            return ()

        lax.fori_loop(0, _GROUPS, do_group, ())
        pltpu.sync_copy(
            rows_b, out_hbm.at[pl.ds(cbase * _EMBED_DIM, _CHUNK_E * _EMBED_DIM)]
        )
        return ()

    lax.fori_loop(0, _N_CHUNKS, do_chunk, ())


def kernel(indices, table):
    idx_flat = indices.reshape(-1)
    table_flat = table.reshape(-1)
    out = _sc_gather(idx_flat, table_flat)
    return out.reshape(_BATCH, _NUM_SETS, _EMBED_DIM)


# speedup vs baseline: 5.6617x; 1.3741x over previous
import functools
import jax
import jax.numpy as jnp
from jax import lax
from jax.experimental import pallas as pl
from jax.experimental.pallas import tpu as pltpu
from jax.experimental.pallas import tpu_sc as plsc

_NUM_SETS = 26
_SET_LEN = 10
_EMBED_DIM = 64
_BATCH = 16384
_NUM_ROWS = _NUM_SETS * _SET_LEN

_L = 16
_NC, _NS = 2, 16
_NW = _NC * _NS
_E_TOTAL = _BATCH * _NUM_SETS
_E_PER_W = _E_TOTAL // _NW
_CHUNK_E = 416
_N_CHUNKS = _E_PER_W // _CHUNK_E
_GROUPS = _CHUNK_E // _L
_CBLK = 8

_mesh = plsc.VectorSubcoreMesh(core_axis_name="c", subcore_axis_name="s")


@functools.partial(
    pl.kernel,
    out_type=jax.ShapeDtypeStruct((_E_TOTAL * _EMBED_DIM,), jnp.float32),
    mesh=_mesh,
    compiler_params=pltpu.CompilerParams(needs_layout_passes=False),
    scratch_types=[
        pltpu.VMEM((_NUM_ROWS * _EMBED_DIM,), jnp.float32),
        pltpu.VMEM((_E_PER_W,), jnp.int32),
        pltpu.VMEM((_CHUNK_E * _EMBED_DIM,), jnp.float32),
        pltpu.VMEM((_CHUNK_E * _EMBED_DIM,), jnp.float32),
        pltpu.SemaphoreType.DMA,
        pltpu.SemaphoreType.DMA,
    ],
)
def _sc_gather(idx_hbm, table_hbm, out_hbm, table_v, idx_v, rows_a, rows_b,
               sem_a, sem_b):
    wid = lax.axis_index("s") * _NC + lax.axis_index("c")
    wbase = wid * _E_PER_W

    pltpu.sync_copy(table_hbm, table_v)
    pltpu.sync_copy(idx_hbm.at[pl.ds(wbase, _E_PER_W)], idx_v)

    lane = lax.iota(jnp.int32, _L)

    def compute_chunk(chunk, rows_buf):
        ebase = chunk * _CHUNK_E

        def do_group(g, _):
            pos = g * _L + lane
            offs = lax.rem(pos, _NUM_SETS) * _SET_LEN
            rows = idx_v[pl.ds(ebase + g * _L, _L)] + offs
            src_base = rows * _EMBED_DIM
            dst_base = pos * _EMBED_DIM
            for c0 in range(0, _EMBED_DIM, _CBLK):
                vs = [
                    plsc.load_gather(table_v, [src_base + (c0 + j)])
                    for j in range(_CBLK)
                ]
                for j in range(_CBLK):
                    plsc.store_scatter(rows_buf, [dst_base + (c0 + j)], vs[j])
            return ()

        lax.fori_loop(0, _GROUPS, do_group, ())

    def start_out(chunk, rows_buf, sem):
        dst = out_hbm.at[
            pl.ds((wbase + chunk * _CHUNK_E) * _EMBED_DIM, _CHUNK_E * _EMBED_DIM)
        ]
        pltpu.async_copy(rows_buf, dst, sem)

    def wait_out(chunk, rows_buf, sem):
        dst = out_hbm.at[
            pl.ds((wbase + chunk * _CHUNK_E) * _EMBED_DIM, _CHUNK_E * _EMBED_DIM)
        ]
        pltpu.make_async_copy(rows_buf, dst, sem).wait()

    compute_chunk(0, rows_a)
    start_out(0, rows_a, sem_a)
    compute_chunk(1, rows_b)
    start_out(1, rows_b, sem_b)

    @pl.loop(2, _N_CHUNKS, step=2)
    def _(k):
        wait_out(k - 2, rows_a, sem_a)
        compute_chunk(k, rows_a)
        start_out(k, rows_a, sem_a)
        wait_out(k - 1, rows_b, sem_b)
        compute_chunk(k + 1, rows_b)
        start_out(k + 1, rows_b, sem_b)

    wait_out(_N_CHUNKS - 2, rows_a, sem_a)
    wait_out(_N_CHUNKS - 1, rows_b, sem_b)


def kernel(indices, table):
    idx_flat = indices.reshape(-1)
    table_flat = table.reshape(-1)
    out = _sc_gather(idx_flat, table_flat)
    return out.reshape(_BATCH, _NUM_SETS, _EMBED_DIM)


# speedup vs baseline: 23.8069x; 4.2049x over previous
import functools
import jax
import jax.numpy as jnp
from jax import lax
from jax.experimental import pallas as pl
from jax.experimental.pallas import tpu as pltpu
from jax.experimental.pallas import tpu_sc as plsc

_NUM_SETS = 26
_SET_LEN = 10
_EMBED_DIM = 64
_BATCH = 16384
_NUM_ROWS = _NUM_SETS * _SET_LEN

_L = 16
_NC, _NS = 2, 16
_NW = _NC * _NS
_R_TOTAL = _NUM_SETS * _EMBED_DIM
_R_PER_W = _R_TOTAL // _NW
_GROUPS = _BATCH // _L

_mesh = plsc.VectorSubcoreMesh(core_axis_name="c", subcore_axis_name="s")


@functools.partial(
    pl.kernel,
    out_type=jax.ShapeDtypeStruct((_NUM_SETS * _EMBED_DIM * _BATCH,), jnp.float32),
    mesh=_mesh,
    compiler_params=pltpu.CompilerParams(needs_layout_passes=False),
    scratch_types=[
        pltpu.VMEM((_NUM_ROWS * _EMBED_DIM,), jnp.float32),
        pltpu.VMEM((2 * _BATCH,), jnp.int32),
        pltpu.VMEM((_BATCH,), jnp.float32),
        pltpu.VMEM((_BATCH,), jnp.float32),
        pltpu.SemaphoreType.DMA,
        pltpu.SemaphoreType.DMA,
    ],
)
def _sc_gather(idx_hbm, table_hbm, out_hbm, table_v, idx_v, row_a, row_b,
               sem_a, sem_b):
    wid = lax.axis_index("s") * _NC + lax.axis_index("c")
    rbase = wid * _R_PER_W

    s0 = rbase // _EMBED_DIM
    s1 = (rbase + _R_PER_W - 1) // _EMBED_DIM

    pltpu.sync_copy(table_hbm, table_v)
    pltpu.sync_copy(idx_hbm.at[pl.ds(s0 * _BATCH, _BATCH)],
                    idx_v.at[pl.ds(0, _BATCH)])
    pltpu.sync_copy(idx_hbm.at[pl.ds(s1 * _BATCH, _BATCH)],
                    idx_v.at[pl.ds(_BATCH, _BATCH)])

    lane = lax.iota(jnp.int32, _L)
    lane_c = jnp.minimum(lane, _SET_LEN - 1)

    def do_row(r, row_buf, sem, drain):
        row = rbase + r
        s = row // _EMBED_DIM
        c = lax.rem(row, _EMBED_DIM)
        if drain:
            pltpu.make_async_copy(
                row_buf, out_hbm.at[pl.ds(row * _BATCH, _BATCH)], sem
            ).wait()
        tvec = plsc.load_gather(
            table_v, [s * (_SET_LEN * _EMBED_DIM) + lane_c * _EMBED_DIM + c]
        )
        ioff = jnp.where(s == s0, 0, _BATCH)

        @pl.loop(0, _GROUPS, step=4, unroll=2)
        def _(g):
            idxs = [idx_v[pl.ds(ioff + (g + k) * _L, _L)] for k in range(4)]
            vals = [
                jnp.take_along_axis(tvec, iv, axis=0, mode="promise_in_bounds")
                for iv in idxs
            ]
            for k in range(4):
                row_buf[pl.ds((g + k) * _L, _L)] = vals[k]

        pltpu.async_copy(
            row_buf, out_hbm.at[pl.ds(row * _BATCH, _BATCH)], sem
        )

    do_row(0, row_a, sem_a, False)
    do_row(1, row_b, sem_b, False)

    @pl.loop(2, _R_PER_W, step=2)
    def _(r):
        do_row(r, row_a, sem_a, True)
        do_row(r + 1, row_b, sem_b, True)

    pltpu.make_async_copy(row_a, out_hbm.at[pl.ds(0, _BATCH)], sem_a).wait()
    pltpu.make_async_copy(row_b, out_hbm.at[pl.ds(0, _BATCH)], sem_b).wait()


def kernel(indices, table):
    idx_t = indices.T.reshape(-1)
    table_flat = table.reshape(-1)
    out = _sc_gather(idx_t, table_flat)
    return out.reshape(_NUM_SETS, _EMBED_DIM, _BATCH).transpose(2, 0, 1)


# speedup vs baseline: 83.0851x; 3.4900x over previous
import functools
import jax
import jax.numpy as jnp
from jax import lax
from jax.experimental import pallas as pl
from jax.experimental.pallas import tpu as pltpu
from jax.experimental.pallas import tpu_sc as plsc

_NUM_SETS = 26
_SET_LEN = 10
_EMBED_DIM = 64
_BATCH = 16384
_NUM_ROWS = _NUM_SETS * _SET_LEN

_L = 16
_NC, _NS = 2, 16
_NW = _NC * _NS
_CT = _EMBED_DIM // 8
_BCH = 4
_TPU_B = _BATCH // (_BCH * 128)
_UNITS = _NUM_SETS * _CT * _BCH
_U_PER_W = _UNITS // _NW

_mesh = plsc.VectorSubcoreMesh(core_axis_name="c", subcore_axis_name="s")


@functools.partial(
    pl.kernel,
    out_type=jax.ShapeDtypeStruct((_NUM_SETS, _EMBED_DIM, _BATCH), jnp.float32),
    mesh=_mesh,
    compiler_params=pltpu.CompilerParams(
        needs_layout_passes=False, use_tc_tiling_on_sc=True
    ),
    scratch_types=[
        pltpu.VMEM((_NUM_ROWS * _EMBED_DIM,), jnp.float32),
        pltpu.VMEM((2 * _BATCH,), jnp.int32),
        pltpu.VMEM((_TPU_B, 8, 128), jnp.float32),
        pltpu.VMEM((_TPU_B, 8, 128), jnp.float32),
        pltpu.SemaphoreType.DMA,
        pltpu.SemaphoreType.DMA,
    ],
)
def _sc_gather(idx_hbm, table_hbm, out_hbm, table_v, idx_v, stg_a, stg_b,
               sem_a, sem_b):
    wid = lax.axis_index("s") * _NC + lax.axis_index("c")
    ubase = wid * _U_PER_W
    s0 = ubase // (_CT * _BCH)

    pltpu.sync_copy(table_hbm, table_v)
    s_last = (ubase + _U_PER_W - 1) // (_CT * _BCH)
    pltpu.sync_copy(idx_hbm.at[pl.ds(s0 * _BATCH, _BATCH)],
                    idx_v.at[pl.ds(0, _BATCH)])
    pltpu.sync_copy(idx_hbm.at[pl.ds(s_last * _BATCH, _BATCH)],
                    idx_v.at[pl.ds(_BATCH, _BATCH)])

    lane = lax.iota(jnp.int32, _L)
    lane_c = jnp.minimum(lane, _SET_LEN - 1)

    def out_tile(s, ct, bt):
        return out_hbm.at[pl.ds(s, 1), pl.ds(ct * 8, 8), pl.ds(bt * 128, 128)]

    def do_unit(ul, stg, sem, drain):
        u = ubase + ul
        s = u // (_CT * _BCH)
        r = lax.rem(u, _CT * _BCH)
        ct = r // _BCH
        ch = lax.rem(r, _BCH)
        ioff = jnp.where(s == s0, 0, _BATCH)
        if drain:
            for t in range(_TPU_B):
                pltpu.make_async_copy(
                    stg.at[pl.ds(t, 1)], out_tile(s, ct, t), sem
                ).wait()
        tvecs = [
            plsc.load_gather(
                table_v,
                [s * (_SET_LEN * _EMBED_DIM) + lane_c * _EMBED_DIM
                 + (ct * 8 + i)],
            )
            for i in range(8)
        ]

        @pl.loop(0, _TPU_B)
        def _(t):
            b0 = ioff + ch * (_TPU_B * 128) + t * 128
            ivs = [idx_v[pl.ds(b0 + g * _L, _L)] for g in range(8)]
            for i in range(8):
                for g in range(8):
                    stg[t, i, pl.ds(g * _L, _L)] = jnp.take_along_axis(
                        tvecs[i], ivs[g], axis=0, mode="promise_in_bounds"
                    )
            pltpu.async_copy(
                stg.at[pl.ds(t, 1)],
                out_tile(s, ct, ch * _TPU_B + t),
                sem,
            )

    do_unit(0, stg_a, sem_a, False)
    do_unit(1, stg_b, sem_b, False)

    @pl.loop(2, _U_PER_W, step=2)
    def _(ul):
        do_unit(ul, stg_a, sem_a, True)
        do_unit(ul + 1, stg_b, sem_b, True)

    for t in range(_TPU_B):
        pltpu.make_async_copy(
            stg_a.at[pl.ds(t, 1)], out_tile(0, 0, t), sem_a
        ).wait()
        pltpu.make_async_copy(
            stg_b.at[pl.ds(t, 1)], out_tile(0, 0, t), sem_b
        ).wait()


def kernel(indices, table):
    idx_t = indices.T.reshape(-1)
    table_flat = table.reshape(-1)
    out = _sc_gather(idx_t, table_flat)
    return out.transpose(2, 0, 1)


# speedup vs baseline: 84.3775x; 1.0156x over previous
import functools
import jax
import jax.numpy as jnp
from jax import lax
from jax.experimental import pallas as pl
from jax.experimental.pallas import tpu as pltpu
from jax.experimental.pallas import tpu_sc as plsc

_NUM_SETS = 26
_SET_LEN = 10
_EMBED_DIM = 64
_BATCH = 16384
_NUM_ROWS = _NUM_SETS * _SET_LEN

_L = 16
_NC, _NS = 2, 16
_NW = _NC * _NS
_CT = _EMBED_DIM // 8
_BCH = 4
_TPU_B = _BATCH // (_BCH * 128)
_UNITS = _NUM_SETS * _CT * _BCH
_U_PER_W = _UNITS // _NW

_mesh = plsc.VectorSubcoreMesh(core_axis_name="c", subcore_axis_name="s")


@functools.partial(
    pl.kernel,
    out_type=jax.ShapeDtypeStruct((_NUM_SETS, _EMBED_DIM, _BATCH), jnp.float32),
    mesh=_mesh,
    compiler_params=pltpu.CompilerParams(
        needs_layout_passes=False, use_tc_tiling_on_sc=True
    ),
    scratch_types=[
        pltpu.VMEM((_NUM_ROWS * _EMBED_DIM,), jnp.float32),
        pltpu.VMEM((2 * _BATCH,), jnp.int32),
        pltpu.VMEM((1, 8, _TPU_B * 128), jnp.float32),
        pltpu.VMEM((1, 8, _TPU_B * 128), jnp.float32),
        pltpu.SemaphoreType.DMA,
        pltpu.SemaphoreType.DMA,
    ],
)
def _sc_gather(idx_hbm, table_hbm, out_hbm, table_v, idx_v, stg_a, stg_b,
               sem_a, sem_b):
    wid = lax.axis_index("s") * _NC + lax.axis_index("c")
    ubase = wid * _U_PER_W
    s0 = ubase // (_CT * _BCH)

    pltpu.sync_copy(table_hbm, table_v)
    s_last = (ubase + _U_PER_W - 1) // (_CT * _BCH)
    pltpu.sync_copy(idx_hbm.at[pl.ds(s0 * _BATCH, _BATCH)],
                    idx_v.at[pl.ds(0, _BATCH)])
    pltpu.sync_copy(idx_hbm.at[pl.ds(s_last * _BATCH, _BATCH)],
                    idx_v.at[pl.ds(_BATCH, _BATCH)])

    lane = lax.iota(jnp.int32, _L)
    lane_c = jnp.minimum(lane, _SET_LEN - 1)

    def out_unit(s, ct, ch):
        return out_hbm.at[
            pl.ds(s, 1), pl.ds(ct * 8, 8), pl.ds(ch * (_TPU_B * 128), _TPU_B * 128)
        ]

    def do_unit(ul, stg, sem, drain):
        u = ubase + ul
        s = u // (_CT * _BCH)
        r = lax.rem(u, _CT * _BCH)
        ct = r // _BCH
        ch = lax.rem(r, _BCH)
        ioff = jnp.where(s == s0, 0, _BATCH)
        if drain:
            pltpu.make_async_copy(stg, out_unit(s, ct, ch), sem).wait()
        tvecs = [
            plsc.load_gather(
                table_v,
                [s * (_SET_LEN * _EMBED_DIM) + lane_c * _EMBED_DIM
                 + (ct * 8 + i)],
            )
            for i in range(8)
        ]

        @pl.loop(0, _TPU_B)
        def _(t):
            b0 = ioff + ch * (_TPU_B * 128) + t * 128
            ivs = [idx_v[pl.ds(b0 + g * _L, _L)] for g in range(8)]
            for i in range(8):
                for g in range(8):
                    stg[0, i, pl.ds(t * 128 + g * _L, _L)] = jnp.take_along_axis(
                        tvecs[i], ivs[g], axis=0, mode="promise_in_bounds"
                    )

        pltpu.async_copy(stg, out_unit(s, ct, ch), sem)

    do_unit(0, stg_a, sem_a, False)
    do_unit(1, stg_b, sem_b, False)

    @pl.loop(2, _U_PER_W, step=2)
    def _(ul):
        do_unit(ul, stg_a, sem_a, True)
        do_unit(ul + 1, stg_b, sem_b, True)

    pltpu.make_async_copy(stg_a, out_unit(0, 0, 0), sem_a).wait()
    pltpu.make_async_copy(stg_b, out_unit(0, 0, 0), sem_b).wait()


def kernel(indices, table):
    idx_t = indices.T.reshape(-1)
    table_flat = table.reshape(-1)
    out = _sc_gather(idx_t, table_flat)
    return out.transpose(2, 0, 1)


# speedup vs baseline: 86.2117x; 1.0217x over previous
import functools
import jax
import jax.numpy as jnp
from jax import lax
from jax.experimental import pallas as pl
from jax.experimental.pallas import tpu as pltpu
from jax.experimental.pallas import tpu_sc as plsc

_NUM_SETS = 26
_SET_LEN = 10
_EMBED_DIM = 64
_BATCH = 16384
_NUM_ROWS = _NUM_SETS * _SET_LEN

_L = 16
_NC, _NS = 2, 16
_NW = _NC * _NS
_CT = _EMBED_DIM // 8
_BCH = 4
_TPU_B = _BATCH // (_BCH * 128)
_UNITS = _NUM_SETS * _CT * _BCH
_U_PER_W = _UNITS // _NW

_mesh = plsc.VectorSubcoreMesh(core_axis_name="c", subcore_axis_name="s")


@functools.partial(
    pl.kernel,
    out_type=jax.ShapeDtypeStruct((_NUM_SETS, _EMBED_DIM, _BATCH), jnp.float32),
    mesh=_mesh,
    compiler_params=pltpu.CompilerParams(
        needs_layout_passes=False, use_tc_tiling_on_sc=True
    ),
    scratch_types=[
        pltpu.VMEM((_NUM_ROWS * _EMBED_DIM,), jnp.float32),
        pltpu.VMEM((2, _BATCH), jnp.int32),
        pltpu.VMEM((1, 8, _TPU_B * 128), jnp.float32),
        pltpu.VMEM((1, 8, _TPU_B * 128), jnp.float32),
        pltpu.SemaphoreType.DMA,
        pltpu.SemaphoreType.DMA,
    ],
)
def _sc_gather(idx_hbm, table_hbm, out_hbm, table_v, idx_v, stg_a, stg_b,
               sem_a, sem_b):
    wid = lax.axis_index("s") * _NC + lax.axis_index("c")
    ubase = wid * _U_PER_W
    s0 = ubase // (_CT * _BCH)

    pltpu.sync_copy(table_hbm, table_v)
    s_last = (ubase + _U_PER_W - 1) // (_CT * _BCH)
    pltpu.sync_copy(idx_hbm.at[pl.ds(s0, 1)], idx_v.at[pl.ds(0, 1)])
    pltpu.sync_copy(idx_hbm.at[pl.ds(s_last, 1)], idx_v.at[pl.ds(1, 1)])

    lane = lax.iota(jnp.int32, _L)
    lane_c = jnp.minimum(lane, _SET_LEN - 1)

    def out_unit(s, ct, ch):
        return out_hbm.at[
            pl.ds(s, 1), pl.ds(ct * 8, 8), pl.ds(ch * (_TPU_B * 128), _TPU_B * 128)
        ]

    def do_unit(ul, stg, sem, drain):
        u = ubase + ul
        s = u // (_CT * _BCH)
        r = lax.rem(u, _CT * _BCH)
        ct = r // _BCH
        ch = lax.rem(r, _BCH)
        irow = jnp.where(s == s0, 0, 1)
        if drain:
            pltpu.make_async_copy(stg, out_unit(s, ct, ch), sem).wait()
        tvecs = [
            plsc.load_gather(
                table_v,
                [s * (_SET_LEN * _EMBED_DIM) + lane_c * _EMBED_DIM
                 + (ct * 8 + i)],
            )
            for i in range(8)
        ]

        @pl.loop(0, _TPU_B)
        def _(t):
            b0 = ch * (_TPU_B * 128) + t * 128
            ivs = [idx_v[irow, pl.ds(b0 + g * _L, _L)] for g in range(8)]
            for i in range(8):
                for g in range(8):
                    stg[0, i, pl.ds(t * 128 + g * _L, _L)] = jnp.take_along_axis(
                        tvecs[i], ivs[g], axis=0, mode="promise_in_bounds"
                    )

        pltpu.async_copy(stg, out_unit(s, ct, ch), sem)

    do_unit(0, stg_a, sem_a, False)
    do_unit(1, stg_b, sem_b, False)

    @pl.loop(2, _U_PER_W, step=2)
    def _(ul):
        do_unit(ul, stg_a, sem_a, True)
        do_unit(ul + 1, stg_b, sem_b, True)

    pltpu.make_async_copy(stg_a, out_unit(0, 0, 0), sem_a).wait()
    pltpu.make_async_copy(stg_b, out_unit(0, 0, 0), sem_b).wait()


def kernel(indices, table):
    idx_t = indices.T
    table_flat = table.reshape(-1)
    out = _sc_gather(idx_t, table_flat)
    return out.transpose(2, 0, 1)
